# baseline (speedup 1.0000x reference)
import functools

import jax
import jax.numpy as jnp
from jax import lax
from jax.experimental import pallas as pl
from jax.experimental.pallas import tpu as pltpu
from jax.experimental.pallas import tpu_sc as plsc

D = 64
B = 4096
L = 200
NC = 2
NS = 16
NW = NC * NS
ROWS = B * L
IR = ROWS // 128
IR_W = IR // NW
KB = 4
GROUPS = IR_W // KB
BPW = B // NW
SCALE = 8.0

_mesh = plsc.VectorSubcoreMesh(core_axis_name="c", subcore_axis_name="s")


@functools.partial(
    pl.kernel,
    out_type=jax.ShapeDtypeStruct((ROWS, D), jnp.float32),
    mesh=_mesh,
    scratch_types=[
        pltpu.VMEM((KB, 128), jnp.int32),
        pltpu.VMEM((KB * 128, D), jnp.float32),
        pltpu.VMEM((128,), jnp.int32),
        pltpu.VMEM((1, 128), jnp.int32),
        pltpu.VMEM((128, D), jnp.float32),
        pltpu.SemaphoreType.DMA,
    ],
    compiler_params=pltpu.CompilerParams(use_tc_tiling_on_sc=False),
)
def _embed(uni_idx, task_idx, task_tab, uni_tab, out,
           idx_v, rows_v, tidx_v, tpos_v, trows_v, sem):
    wid = lax.axis_index("s") * NC + lax.axis_index("c")
    ir0 = wid * IR_W

    def group(g, carry):
        r0 = ir0 + g * KB
        pltpu.sync_copy(uni_idx.at[pl.ds(r0, KB)], idx_v)
        handles = [
            pltpu.async_copy(uni_tab.at[idx_v.at[j]],
                             rows_v.at[pl.ds(j * 128, 128)], sem)
            for j in range(KB)
        ]
        for h in handles:
            h.wait()

        def scale_row(r, c2):
            for c in range(D // 16):
                rows_v[r, pl.ds(c * 16, 16)] = (
                    rows_v[r, pl.ds(c * 16, 16)] * SCALE)
            return c2
        lax.fori_loop(0, KB * 128, scale_row, 0)

        pltpu.sync_copy(rows_v, out.at[pl.ds(r0 * 128, KB * 128)])
        return carry

    lax.fori_loop(0, GROUPS, group, 0)

    b0 = wid * BPW
    for i in range(BPW // 16):
        tpos_v[0, pl.ds(i * 16, 16)] = (
            (b0 + i * 16 + lax.iota(jnp.int32, 16)) * L)
    pltpu.sync_copy(task_idx.at[wid], tidx_v)
    pltpu.async_copy(task_tab.at[tidx_v], trows_v, sem).wait()

    def scale_trow(r, c2):
        for c in range(D // 16):
            trows_v[r, pl.ds(c * 16, 16)] = (
                trows_v[r, pl.ds(c * 16, 16)] * SCALE)
        return c2
    lax.fori_loop(0, BPW, scale_trow, 0)

    pltpu.async_copy(trows_v, out.at[tpos_v.at[0]], sem).wait()


def kernel(task, uni, task_table, uni_table):
    uni_idx = uni.reshape(IR, 128)
    task0 = task[:, 0].reshape(NW, BPW)
    out = _embed(uni_idx, task0, task_table, uni_table)
    return out.reshape(B, L, D)


# speedup vs baseline: 1.1358x; 1.1358x over previous
import functools

import jax
import jax.numpy as jnp
from jax import lax
from jax.experimental import pallas as pl
from jax.experimental.pallas import tpu as pltpu
from jax.experimental.pallas import tpu_sc as plsc

D = 64
B = 4096
L = 200
NC = 2
NS = 16
NW = NC * NS
ROWS = B * L
IR = ROWS // 128
IR_W = IR // NW
KB = 5
GROUPS = IR_W // KB
GR = KB * 128
BPW = B // NW
SCALE = 8.0

_mesh = plsc.VectorSubcoreMesh(core_axis_name="c", subcore_axis_name="s")


@functools.partial(
    pl.kernel,
    out_type=jax.ShapeDtypeStruct((ROWS, D), jnp.float32),
    mesh=_mesh,
    scratch_types=[
        pltpu.VMEM((2, KB, 128), jnp.int32),
        pltpu.VMEM((2, GR, D), jnp.float32),
        pltpu.VMEM((128,), jnp.int32),
        pltpu.VMEM((1, 128), jnp.int32),
        pltpu.VMEM((128, D), jnp.float32),
        pltpu.SemaphoreType.DMA,
        pltpu.SemaphoreType.DMA,
        pltpu.SemaphoreType.DMA,
        pltpu.SemaphoreType.DMA,
        pltpu.SemaphoreType.DMA,
        pltpu.SemaphoreType.DMA,
    ],
    compiler_params=pltpu.CompilerParams(use_tc_tiling_on_sc=False),
)
def _embed(uni_idx, task_idx, task_tab, uni_tab, out,
           idx_v, rows_v, tidx_v, tpos_v, trows_v,
           sg0, sg1, sw0, sw1, si0, si1):
    sg = (sg0, sg1)
    sw = (sw0, sw1)
    si = (si0, si1)
    wid = lax.axis_index("s") * NC + lax.axis_index("c")
    ir0 = wid * IR_W

    def fire_gathers(cb):
        for j in range(KB):
            pltpu.async_copy(uni_tab.at[idx_v.at[cb, j]],
                             rows_v.at[cb, pl.ds(j * 128, 128)], sg[cb])

    def drain_gathers(cb):
        pltpu.make_async_copy(uni_tab.at[pl.ds(0, GR)],
                              rows_v.at[cb], sg[cb]).wait()

    def fire_idx(gg, cb):
        pltpu.async_copy(uni_idx.at[pl.ds(ir0 + gg * KB, KB)],
                         idx_v.at[cb], si[cb])

    def drain_idx(cb):
        pltpu.make_async_copy(uni_idx.at[pl.ds(0, KB)],
                              idx_v.at[cb], si[cb]).wait()

    def fire_write(gg, cb):
        pltpu.async_copy(rows_v.at[cb],
                         out.at[pl.ds((ir0 + gg * KB) * 128, GR)], sw[cb])

    def drain_write(cb):
        pltpu.make_async_copy(rows_v.at[cb],
                              out.at[pl.ds(0, GR)], sw[cb]).wait()

    def scale(cb):
        @plsc.parallel_loop(0, GR, unroll=8)
        def _(r):
            for c in range(D // 16):
                rows_v[cb, r, pl.ds(c * 16, 16)] = (
                    rows_v[cb, r, pl.ds(c * 16, 16)] * SCALE)

    for cb in range(2):
        pltpu.sync_copy(uni_idx.at[pl.ds(ir0 + cb * KB, KB)], idx_v.at[cb])
        fire_gathers(cb)

    @pl.loop(0, GROUPS - 2, step=2)
    def _(g):
        for cb in range(2):
            gg = g + cb
            drain_gathers(cb)
            fire_idx(gg + 2, cb)
            scale(cb)
            fire_write(gg, cb)
            drain_write(cb)
            drain_idx(cb)
            fire_gathers(cb)

    for cb in range(2):
        gg = GROUPS - 2 + cb
        drain_gathers(cb)
        scale(cb)
        fire_write(gg, cb)
    for cb in range(2):
        drain_write(cb)

    b0 = wid * BPW
    for i in range(BPW // 16):
        tpos_v[0, pl.ds(i * 16, 16)] = (
            (b0 + i * 16 + lax.iota(jnp.int32, 16)) * L)
    pltpu.sync_copy(task_idx.at[wid], tidx_v)
    pltpu.async_copy(task_tab.at[tidx_v], trows_v, sg[0]).wait()

    @plsc.parallel_loop(0, BPW, unroll=8)
    def _(r):
        for c in range(D // 16):
            trows_v[r, pl.ds(c * 16, 16)] = (
                trows_v[r, pl.ds(c * 16, 16)] * SCALE)

    pltpu.async_copy(trows_v, out.at[tpos_v.at[0]], sg[0]).wait()


def kernel(task, uni, task_table, uni_table):
    uni_idx = uni.reshape(IR, 128)
    task0 = task[:, 0].reshape(NW, BPW)
    out = _embed(uni_idx, task0, task_table, uni_table)
    return out.reshape(B, L, D)
